# baseline (speedup 1.0000x reference)
import jax
import jax.numpy as jnp
from jax import lax
from jax.experimental import pallas as pl
from jax.experimental.pallas import tpu as pltpu
from jax.experimental.pallas import tpu_sc as plsc

NC, NS = 2, 16
NW = NC * NS
R2 = 8192
D2 = 2048
D = 1024
QPW = R2 // NW
CHUNK = 16
NCHUNK = QPW // CHUNK
NBUF = 3
LEAD = NBUF - 2


def _sc_body(x2_hbm, out_hbm, sbuf, *sems):
    sid = lax.axis_index("s")
    wid = sid * NC + lax.axis_index("c")
    buf = sbuf.at[sid]
    q0 = wid * QPW
    b = wid // 8
    j0 = (wid % 8) * QPW
    dst0 = b * 4096 + j0
    dst1 = b * 4096 + 2048 + j0

    gsems = sems[:NBUF]
    psems = sems[NBUF:]

    def gather(k):
        s = k % NBUF
        return pltpu.async_copy(
            x2_hbm.at[pl.ds(q0 + k * CHUNK, CHUNK)], buf.at[s], gsems[s])

    def put(k, h):
        s = k % NBUF
        return pltpu.async_copy(
            buf.at[s, :, pl.ds(h * D, D)],
            out_hbm.at[pl.ds((dst1 if h else dst0) + k * CHUNK, CHUNK)],
            psems[2 * s + h])

    gd = [None] * NCHUNK
    pd = [None] * NCHUNK
    for k in range(min(LEAD, NCHUNK)):
        gd[k] = gather(k)
    for k in range(NCHUNK):
        j = k + LEAD
        if j < NCHUNK:
            if j >= NBUF:
                for d in pd[j - NBUF]:
                    d.wait()
            gd[j] = gather(j)
        gd[k].wait()
        pd[k] = (put(k, 0), put(k, 1))
    for k in range(max(0, NCHUNK - NBUF), NCHUNK):
        for d in pd[k]:
            d.wait()


def _shuffle_sc(x2):
    mesh = plsc.VectorSubcoreMesh(core_axis_name="c", subcore_axis_name="s")
    return pl.kernel(
        _sc_body,
        out_type=jax.ShapeDtypeStruct((2 * R2, D), jnp.float32),
        mesh=mesh,
        scratch_types=[pltpu.VMEM_SHARED((NS, NBUF, CHUNK, D2), jnp.float32)]
        + [pltpu.SemaphoreType.DMA] * (3 * NBUF),
    )(x2)


def kernel(x):
    B, L, F = x.shape
    x2 = x.reshape(B * L // 2, 2 * F)
    out = _shuffle_sc(x2)
    return out.reshape(B, L, F)
